# baseline (speedup 1.0000x reference)
import functools
import jax
import jax.numpy as jnp
from jax import lax
from jax.experimental import pallas as pl
from jax.experimental.pallas import tpu as pltpu
from jax.experimental.pallas import tpu_sc as plsc

L = 16
NC, NS = 2, 16
NW = NC * NS
CB = 128


def _perm(v, idx):
    return jnp.take_along_axis(v, idx, axis=0)


def _bf16r(x):
    xi = lax.bitcast_convert_type(x, jnp.uint32)
    one = jnp.uint32(1)
    r = xi + jnp.uint32(0x7FFF) + (lax.shift_right_logical(xi, jnp.uint32(16)) & one)
    r = r & jnp.uint32(0xFFFF0000)
    return lax.bitcast_convert_type(r, jnp.float32)


def _build(N, B, C, H, W, D):
    nfull = N // CB
    tail = N - nfull * CB
    iters = (nfull + NW - 1) // NW
    mesh = plsc.VectorSubcoreMesh(core_axis_name="c", subcore_axis_name="s")

    @functools.partial(
        pl.kernel,
        mesh=mesh,
        out_type=jax.ShapeDtypeStruct((N, D + C), jnp.float32),
        compiler_params=pltpu.CompilerParams(use_tc_tiling_on_sc=False),
        scratch_types=[
            pltpu.VMEM((64,), jnp.float32),
            pltpu.VMEM((64,), jnp.float32),
            pltpu.VMEM((64,), jnp.float32),
            pltpu.VMEM((2, 6, CB), jnp.int32),
            pltpu.VMEM((2, 4, CB), jnp.int32),
            pltpu.VMEM((2, 4, CB), jnp.float32),
            pltpu.VMEM((2, 4, CB, C // 2), jnp.int32),
            pltpu.VMEM((2, CB, C), jnp.float32),
            pltpu.SemaphoreType.DMA,
            pltpu.SemaphoreType.DMA,
            pltpu.SemaphoreType.DMA,
        ],
    )
    def k(vf, packed, tbl, p2, rc, tv, out,
          p2_v, rc_v, tv_v, in_v, idx_v, w_v, r_v, m_v,
          sem_i, sem_g, sem_o):
        wid = lax.axis_index("s") * NC + lax.axis_index("c")

        h1 = pltpu.async_copy(p2, p2_v, sem_i)
        h2 = pltpu.async_copy(rc, rc_v, sem_i)
        h3 = pltpu.async_copy(tv, tv_v, sem_i)
        h1.wait(); h2.wait(); h3.wait()

        lane = lax.iota(jnp.int32, L)
        li4 = lax.shift_right_logical(lane, 2) * 4
        lj = lax.bitwise_and(lane, 3)
        projvecs = [jnp.zeros((L,), jnp.float32) for _ in range(12)]
        for b in range(B):
            mp2 = _bf16r(p2_v[pl.ds(b * 16, L)])
            mrc = _bf16r(rc_v[pl.ds(b * 16, L)])
            mtv = _bf16r(tv_v[pl.ds(b * 16, L)])
            t1 = jnp.zeros((L,), jnp.float32)
            for kk in range(4):
                t1 = t1 + _perm(mp2, li4 + kk) * _perm(mrc, kk * 4 + lj)
            t1 = _bf16r(t1)
            pr = jnp.zeros((L,), jnp.float32)
            for kk in range(4):
                pr = pr + _perm(t1, li4 + kk) * _perm(mtv, kk * 4 + lj)
            for j in range(12):
                projvecs[j] = jnp.where(lane == b, pr[j], projvecs[j])

        def in_copy(vbase, par, size):
            return (
                pltpu.make_async_copy(packed.at[:, pl.ds(vbase, size)],
                                      in_v.at[par, :, pl.ds(0, size)], sem_i),
            )

        def gather_copies(par, size):
            return [
                pltpu.make_async_copy(tbl.at[idx_v.at[par, kk, pl.ds(0, size)]],
                                      r_v.at[par, kk, pl.ds(0, size), :], sem_g)
                for kk in range(4)
            ]

        def out_copy(vbase, par, size):
            return (
                pltpu.make_async_copy(m_v.at[par, pl.ds(0, size), :],
                                      out.at[pl.ds(vbase, size), pl.ds(D, C)],
                                      sem_o),
                pltpu.make_async_copy(vf.at[pl.ds(vbase, size), :],
                                      out.at[pl.ds(vbase, size), pl.ds(0, D)],
                                      sem_o),
            )

        def compute_idx(par, size):
            for gi in range(size // L):
                s = pl.ds(gi * L, L)
                b16 = in_v[par, 3, s]
                zf = in_v[par, 0, s].astype(jnp.float32) * 0.1 + (-3.0)
                yf = in_v[par, 1, s].astype(jnp.float32) * 0.05 + (-40.0)
                xf = in_v[par, 2, s].astype(jnp.float32) * 0.05
                ruv = lax.bitcast_convert_type(in_v[par, 4, s], jnp.float32)
                rvv = lax.bitcast_convert_type(in_v[par, 5, s], jnp.float32)
                cc = [_perm(pv, b16) for pv in projvecs]
                u_raw = cc[0] * xf + cc[1] * yf + cc[2] * zf + cc[3]
                v_raw = cc[4] * xf + cc[5] * yf + cc[6] * zf + cc[7]
                dep = cc[8] * xf + cc[9] * yf + cc[10] * zf + cc[11]
                sd = jnp.where(jnp.abs(dep) < 1e-3, jnp.float32(1e-3), dep)
                u = jnp.clip(u_raw / sd + ruv, 0.0, float(W - 1))
                v = jnp.clip(v_raw / sd + rvv, 0.0, float(H - 1))
                u0 = u.astype(jnp.int32)
                v0 = v.astype(jnp.int32)
                u1 = jnp.minimum(u0 + 1, W - 1)
                v1 = jnp.minimum(v0 + 1, H - 1)
                wu = u - u0.astype(jnp.float32)
                wv = v - v0.astype(jnp.float32)
                row0 = b16 * (H * W) + v0 * W
                row1 = b16 * (H * W) + v1 * W
                idx_v[par, 0, s] = row0 + u0
                idx_v[par, 1, s] = row0 + u1
                idx_v[par, 2, s] = row1 + u0
                idx_v[par, 3, s] = row1 + u1
                w_v[par, 0, s] = (1.0 - wv) * (1.0 - wu)
                w_v[par, 1, s] = (1.0 - wv) * wu
                w_v[par, 2, s] = wv * (1.0 - wu)
                w_v[par, 3, s] = wv * wu

        def comb(par, size):
            def body(gi, _):
                gbase = gi * L
                gs = pl.ds(gbase, L)
                w00v = w_v[par, 0, gs]
                w01v = w_v[par, 1, gs]
                w10v = w_v[par, 2, gs]
                w11v = w_v[par, 3, gs]
                mhi = jnp.int32(-65536)
                for j in range(L):
                    i = gbase + j
                    w00 = w00v[j]
                    w01 = w01v[j]
                    w10 = w10v[j]
                    w11 = w11v[j]
                    for half in range(C // 32):
                        cs = pl.ds(half * L, L)
                        ws = [r_v[par, kk, i, cs] for kk in range(4)]
                        lo = [lax.bitcast_convert_type(
                            lax.shift_left(w, jnp.int32(16)), jnp.float32)
                            for w in ws]
                        hi = [lax.bitcast_convert_type(w & mhi, jnp.float32)
                              for w in ws]
                        m_v[par, i, pl.ds(half * 32, L)] = (
                            lo[0] * w00 + lo[1] * w01 + lo[2] * w10 + lo[3] * w11)
                        m_v[par, i, pl.ds(half * 32 + L, L)] = (
                            hi[0] * w00 + hi[1] * w01 + hi[2] * w10 + hi[3] * w11)
                return 0

            lax.fori_loop(0, size // L, body, 0)

        @pl.when(wid < nfull)
        def _():
            for h in in_copy(wid * CB, 0, CB):
                h.start()

        def step(kstep, _):
            for par in (0, 1):
                kk = kstep * 2 + par
                j = wid + kk * NW

                @pl.when(j < nfull)
                def _():
                    for h in in_copy(j * CB, par, CB):
                        h.wait()
                    compute_idx(par, CB)
                    for h in gather_copies(par, CB):
                        h.start()

                jp = j - NW

                @pl.when((jp >= 0) & (jp < nfull))
                def _():
                    jo = jp - 2 * NW
                    @pl.when(jo >= 0)
                    def _():
                        for h in out_copy(jo * CB, 1 - par, CB):
                            h.wait()
                    for h in gather_copies(1 - par, CB):
                        h.wait()
                    comb(1 - par, CB)
                    for h in out_copy(jp * CB, 1 - par, CB):
                        h.start()

                jn = j + NW

                @pl.when(jn < nfull)
                def _():
                    for h in in_copy(jn * CB, 1 - par, CB):
                        h.start()

            return 0

        lax.fori_loop(0, (iters + 2) // 2, step, 0)

        for q in (iters - 3, iters - 2, iters - 1):
            jq = wid + q * NW

            @pl.when((jq < nfull) & (jq + 2 * NW >= nfull))
            def _():
                for h in out_copy(jq * CB, q & 1, CB):
                    h.wait()

        if tail:
            @pl.when(wid == NW - 1)
            def _():
                for h in in_copy(nfull * CB, 0, tail):
                    h.start()
                for h in in_copy(nfull * CB, 0, tail):
                    h.wait()
                compute_idx(0, tail)
                for h in gather_copies(0, tail):
                    h.start()
                for h in gather_copies(0, tail):
                    h.wait()
                comb(0, tail)
                for h in out_copy(nfull * CB, 0, tail):
                    h.start()
                for h in out_copy(nfull * CB, 0, tail):
                    h.wait()

    return k


def kernel(voxel_features, voxel_coords, batch_idx, rgb_features, P2, Trv2c,
           rect, pixel_refine):
    N, D = voxel_features.shape
    B, C, H, W = rgb_features.shape
    R = B * H * W
    t = jnp.transpose(rgb_features, (0, 2, 3, 1)).reshape(R, C)
    t = t.reshape(R, C // 32, 2, L).transpose(0, 1, 3, 2)
    tbl = lax.bitcast_convert_type(
        t.astype(jnp.bfloat16).reshape(R, C // 2, 2), jnp.int32)
    packed = jnp.stack([
        voxel_coords[:, 0], voxel_coords[:, 1], voxel_coords[:, 2],
        batch_idx,
        lax.bitcast_convert_type(pixel_refine[:, 0], jnp.int32),
        lax.bitcast_convert_type(pixel_refine[:, 1], jnp.int32),
    ])
    k = _build(N, B, C, H, W, D)
    return k(voxel_features, packed, tbl,
             P2.reshape(-1), rect.reshape(-1), Trv2c.reshape(-1))


# speedup vs baseline: 1.2059x; 1.2059x over previous
import functools
import jax
import jax.numpy as jnp
from jax import lax
from jax.experimental import pallas as pl
from jax.experimental.pallas import tpu as pltpu
from jax.experimental.pallas import tpu_sc as plsc

L = 16
NC, NS = 2, 16
NW = NC * NS
CB = 128


def _perm(v, idx):
    return jnp.take_along_axis(v, idx, axis=0)


def _bf16r(x):
    xi = lax.bitcast_convert_type(x, jnp.uint32)
    one = jnp.uint32(1)
    r = xi + jnp.uint32(0x7FFF) + (lax.shift_right_logical(xi, jnp.uint32(16)) & one)
    r = r & jnp.uint32(0xFFFF0000)
    return lax.bitcast_convert_type(r, jnp.float32)


def _build(N, B, C, H, W, D):
    nfull = N // CB
    tail = N - nfull * CB
    iters = (nfull + NW - 1) // NW
    mesh = plsc.VectorSubcoreMesh(core_axis_name="c", subcore_axis_name="s")

    @functools.partial(
        pl.kernel,
        mesh=mesh,
        out_type=jax.ShapeDtypeStruct((N, D + C), jnp.float32),
        compiler_params=pltpu.CompilerParams(use_tc_tiling_on_sc=False),
        scratch_types=[
            pltpu.VMEM((64,), jnp.float32),
            pltpu.VMEM((64,), jnp.float32),
            pltpu.VMEM((64,), jnp.float32),
            pltpu.VMEM((2, 6, CB), jnp.int32),
            pltpu.VMEM((2, 4, CB), jnp.int32),
            pltpu.VMEM((2, 4, CB), jnp.float32),
            pltpu.VMEM((2, 4, CB, C // 2), jnp.int32),
            pltpu.VMEM((2, CB, D), jnp.float32),
            pltpu.VMEM((2, CB, D + C), jnp.float32),
            pltpu.SemaphoreType.DMA,
            pltpu.SemaphoreType.DMA,
            pltpu.SemaphoreType.DMA,
        ],
    )
    def k(vf, packed, tbl, p2, rc, tv, out,
          p2_v, rc_v, tv_v, in_v, idx_v, w_v, r_v, vf_v, m_v,
          sem_i, sem_g, sem_o):
        wid = lax.axis_index("s") * NC + lax.axis_index("c")

        h1 = pltpu.async_copy(p2, p2_v, sem_i)
        h2 = pltpu.async_copy(rc, rc_v, sem_i)
        h3 = pltpu.async_copy(tv, tv_v, sem_i)
        h1.wait(); h2.wait(); h3.wait()

        lane = lax.iota(jnp.int32, L)
        li4 = lax.shift_right_logical(lane, 2) * 4
        lj = lax.bitwise_and(lane, 3)
        projvecs = [jnp.zeros((L,), jnp.float32) for _ in range(12)]
        for b in range(B):
            mp2 = _bf16r(p2_v[pl.ds(b * 16, L)])
            mrc = _bf16r(rc_v[pl.ds(b * 16, L)])
            mtv = _bf16r(tv_v[pl.ds(b * 16, L)])
            t1 = jnp.zeros((L,), jnp.float32)
            for kk in range(4):
                t1 = t1 + _perm(mp2, li4 + kk) * _perm(mrc, kk * 4 + lj)
            t1 = _bf16r(t1)
            pr = jnp.zeros((L,), jnp.float32)
            for kk in range(4):
                pr = pr + _perm(t1, li4 + kk) * _perm(mtv, kk * 4 + lj)
            for j in range(12):
                projvecs[j] = jnp.where(lane == b, pr[j], projvecs[j])

        def in_copy(vbase, par, size):
            return (
                pltpu.make_async_copy(packed.at[:, pl.ds(vbase, size)],
                                      in_v.at[par, :, pl.ds(0, size)], sem_i),
                pltpu.make_async_copy(vf.at[pl.ds(vbase, size), :],
                                      vf_v.at[par, pl.ds(0, size), :], sem_i),
            )

        def gather_copies(par, size):
            return [
                pltpu.make_async_copy(tbl.at[idx_v.at[par, kk, pl.ds(0, size)]],
                                      r_v.at[par, kk, pl.ds(0, size), :], sem_g)
                for kk in range(4)
            ]

        def out_copy(vbase, par, size):
            return pltpu.make_async_copy(m_v.at[par, pl.ds(0, size), :],
                                         out.at[pl.ds(vbase, size), :], sem_o)

        def compute_idx(par, size):
            for gi in range(size // L):
                s = pl.ds(gi * L, L)
                b16 = in_v[par, 3, s]
                zf = in_v[par, 0, s].astype(jnp.float32) * 0.1 + (-3.0)
                yf = in_v[par, 1, s].astype(jnp.float32) * 0.05 + (-40.0)
                xf = in_v[par, 2, s].astype(jnp.float32) * 0.05
                ruv = lax.bitcast_convert_type(in_v[par, 4, s], jnp.float32)
                rvv = lax.bitcast_convert_type(in_v[par, 5, s], jnp.float32)
                cc = [_perm(pv, b16) for pv in projvecs]
                u_raw = cc[0] * xf + cc[1] * yf + cc[2] * zf + cc[3]
                v_raw = cc[4] * xf + cc[5] * yf + cc[6] * zf + cc[7]
                dep = cc[8] * xf + cc[9] * yf + cc[10] * zf + cc[11]
                sd = jnp.where(jnp.abs(dep) < 1e-3, jnp.float32(1e-3), dep)
                u = jnp.clip(u_raw / sd + ruv, 0.0, float(W - 1))
                v = jnp.clip(v_raw / sd + rvv, 0.0, float(H - 1))
                u0 = u.astype(jnp.int32)
                v0 = v.astype(jnp.int32)
                u1 = jnp.minimum(u0 + 1, W - 1)
                v1 = jnp.minimum(v0 + 1, H - 1)
                wu = u - u0.astype(jnp.float32)
                wv = v - v0.astype(jnp.float32)
                row0 = b16 * (H * W) + v0 * W
                row1 = b16 * (H * W) + v1 * W
                idx_v[par, 0, s] = row0 + u0
                idx_v[par, 1, s] = row0 + u1
                idx_v[par, 2, s] = row1 + u0
                idx_v[par, 3, s] = row1 + u1
                w_v[par, 0, s] = (1.0 - wv) * (1.0 - wu)
                w_v[par, 1, s] = (1.0 - wv) * wu
                w_v[par, 2, s] = wv * (1.0 - wu)
                w_v[par, 3, s] = wv * wu

        def comb(par, size):
            def body(gi, _):
                gbase = gi * L
                gs = pl.ds(gbase, L)
                w00v = w_v[par, 0, gs]
                w01v = w_v[par, 1, gs]
                w10v = w_v[par, 2, gs]
                w11v = w_v[par, 3, gs]
                mhi = jnp.int32(-65536)
                for j in range(L):
                    i = gbase + j
                    w00 = w00v[j]
                    w01 = w01v[j]
                    w10 = w10v[j]
                    w11 = w11v[j]
                    for c4 in range(D // L):
                        cs = pl.ds(c4 * L, L)
                        m_v[par, i, cs] = vf_v[par, i, cs]
                    for half in range(C // 32):
                        cs = pl.ds(half * L, L)
                        ws = [r_v[par, kk, i, cs] for kk in range(4)]
                        lo = [lax.bitcast_convert_type(
                            lax.shift_left(w, jnp.int32(16)), jnp.float32)
                            for w in ws]
                        hi = [lax.bitcast_convert_type(w & mhi, jnp.float32)
                              for w in ws]
                        m_v[par, i, pl.ds(D + half * 32, L)] = (
                            lo[0] * w00 + lo[1] * w01 + lo[2] * w10 + lo[3] * w11)
                        m_v[par, i, pl.ds(D + half * 32 + L, L)] = (
                            hi[0] * w00 + hi[1] * w01 + hi[2] * w10 + hi[3] * w11)
                return 0

            lax.fori_loop(0, size // L, body, 0)

        @pl.when(wid < nfull)
        def _():
            for h in in_copy(wid * CB, 0, CB):
                h.start()

        def step(kstep, _):
            for par in (0, 1):
                kk = kstep * 2 + par
                j = wid + kk * NW

                @pl.when(j < nfull)
                def _():
                    for h in in_copy(j * CB, par, CB):
                        h.wait()
                    compute_idx(par, CB)
                    for h in gather_copies(par, CB):
                        h.start()

                jp = j - NW

                @pl.when((jp >= 0) & (jp < nfull))
                def _():
                    jo = jp - 2 * NW
                    @pl.when(jo >= 0)
                    def _():
                        out_copy(jo * CB, 1 - par, CB).wait()
                    for h in gather_copies(1 - par, CB):
                        h.wait()
                    comb(1 - par, CB)
                    out_copy(jp * CB, 1 - par, CB).start()

                jn = j + NW

                @pl.when(jn < nfull)
                def _():
                    for h in in_copy(jn * CB, 1 - par, CB):
                        h.start()

            return 0

        lax.fori_loop(0, (iters + 2) // 2, step, 0)

        for q in (iters - 3, iters - 2, iters - 1):
            jq = wid + q * NW

            @pl.when((jq < nfull) & (jq + 2 * NW >= nfull))
            def _():
                out_copy(jq * CB, q & 1, CB).wait()

        if tail:
            @pl.when(wid == NW - 1)
            def _():
                for h in in_copy(nfull * CB, 0, tail):
                    h.start()
                for h in in_copy(nfull * CB, 0, tail):
                    h.wait()
                compute_idx(0, tail)
                for h in gather_copies(0, tail):
                    h.start()
                for h in gather_copies(0, tail):
                    h.wait()
                comb(0, tail)
                oc = out_copy(nfull * CB, 0, tail)
                oc.start()
                oc.wait()

    return k


def kernel(voxel_features, voxel_coords, batch_idx, rgb_features, P2, Trv2c,
           rect, pixel_refine):
    N, D = voxel_features.shape
    B, C, H, W = rgb_features.shape
    R = B * H * W
    t = jnp.transpose(rgb_features, (0, 2, 3, 1)).reshape(R, C)
    t = t.reshape(R, C // 32, 2, L).transpose(0, 1, 3, 2)
    tbl = lax.bitcast_convert_type(
        t.astype(jnp.bfloat16).reshape(R, C // 2, 2), jnp.int32)
    packed = jnp.stack([
        voxel_coords[:, 0], voxel_coords[:, 1], voxel_coords[:, 2],
        batch_idx,
        lax.bitcast_convert_type(pixel_refine[:, 0], jnp.int32),
        lax.bitcast_convert_type(pixel_refine[:, 1], jnp.int32),
    ])
    k = _build(N, B, C, H, W, D)
    return k(voxel_features, packed, tbl,
             P2.reshape(-1), rect.reshape(-1), Trv2c.reshape(-1))
